# baseline (speedup 1.0000x reference)
import functools

import jax
import jax.numpy as jnp
import numpy as np
from jax import lax
from jax.experimental import pallas as pl
from jax.experimental.pallas import tpu as pltpu
from jax.experimental.pallas import tpu_sc as plsc

N_NODES = 10000
N_PAD = 10240
D_IN = 128
D_OUT = 128
D_HALF = D_OUT // 2
N_SUBCORES = 16
N_CORES = 2
CHUNK = 128
IB = 8
ROWS_PER_S = N_PAD // N_SUBCORES
STAGE_STEPS = ROWS_PER_S // CHUNK

_PERM = np.empty(D_OUT, np.int32)
for _c in range(N_CORES):
    for _t in range(2):
        _base = D_HALF * _c + 32 * _t
        for _i in range(16):
            _PERM[_base + 2 * _i] = _base + _i
            _PERM[_base + 2 * _i + 1] = _base + 16 + _i


def _mm_body(x_ref, w_ref, o_ref):
    h = jnp.dot(x_ref[...], w_ref[...], preferred_element_type=jnp.float32)
    hb = h.astype(jnp.bfloat16)
    o_ref[0] = hb[:, :D_HALF]
    o_ref[1] = hb[:, D_HALF:]


def _matmul_split(x, W):
    m_blk = 1024
    grid = (x.shape[0] // m_blk,)
    return pl.pallas_call(
        _mm_body,
        grid=grid,
        in_specs=[
            pl.BlockSpec((m_blk, D_IN), lambda i: (i, 0)),
            pl.BlockSpec((D_IN, D_OUT), lambda i: (0, 0)),
        ],
        out_specs=pl.BlockSpec((N_CORES, m_blk, D_HALF), lambda i: (0, i, 0)),
        out_shape=jax.ShapeDtypeStruct(
            (N_CORES, x.shape[0], D_HALF), jnp.bfloat16),
    )(x, W)


def _make_sc_kernel(n_blocks):
    mesh = plsc.VectorSubcoreMesh(core_axis_name="c", subcore_axis_name="s")

    @functools.partial(
        pl.kernel,
        out_type=jax.ShapeDtypeStruct((N_PAD, D_OUT), jnp.float32),
        mesh=mesh,
        scratch_types=[
            pltpu.VMEM_SHARED((N_PAD, D_HALF), jnp.bfloat16),
            pltpu.VMEM_SHARED((N_PAD, D_HALF), jnp.float32),
            pltpu.VMEM((CHUNK, D_HALF), jnp.bfloat16),
            pltpu.VMEM((CHUNK, D_HALF), jnp.bfloat16),
            pltpu.VMEM((CHUNK, D_HALF), jnp.bfloat16),
            pltpu.VMEM((CHUNK, D_HALF), jnp.bfloat16),
            pltpu.VMEM((CHUNK, D_HALF), jnp.float32),
            pltpu.VMEM((CHUNK, D_HALF), jnp.float32),
            pltpu.VMEM((2, IB, CHUNK), jnp.int32),
            pltpu.VMEM((2, IB, CHUNK), jnp.int32),
            pltpu.VMEM((2, IB * CHUNK), jnp.float32),
            [pltpu.SemaphoreType.DMA] * 4,
            [pltpu.SemaphoreType.DMA] * 2,
            pltpu.SemaphoreType.DMA,
            [pltpu.SemaphoreType.DMA] * 2,
        ],
        compiler_params=pltpu.CompilerParams(
            needs_layout_passes=False, use_tc_tiling_on_sc=False),
    )
    def sc_kernel(h_hbm, src_hbm, dst_hbm, val_hbm, out_hbm,
                  h_sp, acc, gba, gbb, gbc, gbd, sba, sbb, sidx, didx, vval,
                  gsems, ssems, isem, hsems):
        c = lax.axis_index("c")
        s = lax.axis_index("s")
        row0 = s * ROWS_PER_S
        gbufs = (gba, gbb, gbc, gbd)
        sbufs = (sba, sbb)
        h_c = h_hbm.at[c]

        def _hsl(i):
            return pl.ds(row0 + i * CHUNK, CHUNK)

        i0 = [
            pltpu.async_copy(src_hbm.at[s].at[pl.ds(0, IB)], sidx.at[0], isem),
            pltpu.async_copy(dst_hbm.at[s].at[pl.ds(0, IB)], didx.at[0], isem),
            pltpu.async_copy(
                val_hbm.at[s].at[pl.ds(0, IB * CHUNK)], vval.at[0], isem),
        ]

        @pl.loop(0, CHUNK)
        def _(r):
            for q in range(D_HALF // 16):
                sba[r, pl.ds(q * 16, 16)] = jnp.zeros((16,), jnp.float32)

        zcp = [
            pltpu.async_copy(sba, acc.at[_hsl(i)], ssems[i % 2])
            for i in range(STAGE_STEPS)
        ]

        hb = (gba, gbb)
        hi = [None] * STAGE_STEPS
        ho = [None] * STAGE_STEPS
        hi[0] = pltpu.async_copy(h_c.at[_hsl(0)], gba, hsems[0])
        for i in range(STAGE_STEPS):
            if i + 1 < STAGE_STEPS:
                if i >= 1:
                    ho[i - 1].wait()
                hi[i + 1] = pltpu.async_copy(
                    h_c.at[_hsl(i + 1)], hb[(i + 1) % 2], hsems[(i + 1) % 2])
            hi[i].wait()
            ho[i] = pltpu.async_copy(hb[i % 2], h_sp.at[_hsl(i)], hsems[i % 2])
        ho[STAGE_STEPS - 2].wait()
        ho[STAGE_STEPS - 1].wait()

        for cp in zcp + i0:
            cp.wait()
        plsc.subcore_barrier()

        def _scale(gb, sb, vv_ref, u):
            @pl.loop(0, CHUNK // 16, unroll=2)
            def _(g):
                vv = vv_ref[pl.ds(u * CHUNK + g * 16, 16)]
                for k in range(16):
                    vsp = jnp.broadcast_to(vv[k], (16,))
                    j = g * 16 + k
                    for t in range(2):
                        seg = gb[j, pl.ds(32 * t, 32)]
                        lo, hi_ = plsc.unpack(
                            seg, format=plsc.PackFormat.INTERLEAVED)
                        sb[j, pl.ds(32 * t, 16)] = lo * vsp
                        sb[j, pl.ds(32 * t + 16, 16)] = hi_ * vsp

        @pl.loop(0, n_blocks)
        def _(b):
            par = b % 2
            nxt = (b + 1) % n_blocks
            sidx_b = sidx.at[par]
            didx_b = didx.at[par]
            vval_b = vval.at[par]

            pf = [
                pltpu.async_copy(
                    src_hbm.at[s].at[pl.ds(nxt * IB, IB)],
                    sidx.at[1 - par], isem),
                pltpu.async_copy(
                    dst_hbm.at[s].at[pl.ds(nxt * IB, IB)],
                    didx.at[1 - par], isem),
                pltpu.async_copy(
                    val_hbm.at[s].at[pl.ds(nxt * IB * CHUNK, IB * CHUNK)],
                    vval.at[1 - par], isem),
            ]

            def _gather(u, i):
                return pltpu.async_copy(
                    h_sp.at[sidx_b.at[u]], gbufs[i], gsems[i])

            def _scatter(u, i):
                return pltpu.async_copy(
                    sbufs[i], acc.at[didx_b.at[u]], ssems[i], add=True)

            gd_ = [None] * IB
            sd_ = [None] * IB
            for u in range(4):
                gd_[u] = _gather(u, u)
            for u in range(IB):
                gi = u % 4
                si = u % 2
                gd_[u].wait()
                if u >= 2:
                    sd_[u - 2].wait()
                _scale(gbufs[gi], sbufs[si], vval_b, u)
                sd_[u] = _scatter(u, si)
                if u + 4 < IB:
                    gd_[u + 4] = _gather(u + 4, gi)
            sd_[IB - 2].wait()
            sd_[IB - 1].wait()
            for cp in pf:
                cp.wait()

        plsc.subcore_barrier()

        def _cin(i):
            return pltpu.async_copy(acc.at[_hsl(i)], sbufs[i % 2], gsems[i % 2])

        def _cout(i):
            return pltpu.async_copy(
                sbufs[i % 2],
                out_hbm.at[pl.ds(row0 + i * CHUNK, CHUNK),
                           pl.ds(c * D_HALF, D_HALF)],
                ssems[i % 2])

        def _relu(i):
            rb = sbufs[i % 2]

            @pl.loop(0, CHUNK)
            def _(r):
                for q in range(D_HALF // 16):
                    sl = pl.ds(q * 16, 16)
                    rb[r, sl] = jnp.maximum(rb[r, sl], 0.0)

        cin = [None] * STAGE_STEPS
        cout = [None] * STAGE_STEPS
        cin[0] = _cin(0)
        for i in range(STAGE_STEPS):
            if i + 1 < STAGE_STEPS:
                if i >= 1:
                    cout[i - 1].wait()
                cin[i + 1] = _cin(i + 1)
            cin[i].wait()
            _relu(i)
            cout[i] = _cout(i)
        cout[STAGE_STEPS - 2].wait()
        cout[STAGE_STEPS - 1].wait()

    return sc_kernel


def kernel(x, edge_index, adj_values, W):
    src = edge_index[0].astype(jnp.int32)
    dst = edge_index[1].astype(jnp.int32)
    val = adj_values.astype(jnp.float32)

    n_edges = src.shape[0]
    blk = N_SUBCORES * CHUNK * IB
    per_s = (-(-n_edges // blk)) * blk // N_SUBCORES
    n_blocks = per_s // (CHUNK * IB)
    pad = per_s * N_SUBCORES - n_edges
    srcp = jnp.pad(src, (0, pad)).reshape(N_SUBCORES, per_s // CHUNK, CHUNK)
    dstp = jnp.pad(dst, (0, pad)).reshape(N_SUBCORES, per_s // CHUNK, CHUNK)
    valp = jnp.pad(val, (0, pad)).reshape(N_SUBCORES, per_s)

    xp = jnp.pad(x, ((0, N_PAD - x.shape[0]), (0, 0)))
    h2 = _matmul_split(xp, W[:, _PERM])
    o = _make_sc_kernel(n_blocks)(h2, srcp, dstp, valp)
    return o[:x.shape[0]]


# speedup vs baseline: 1.4376x; 1.4376x over previous
import functools

import jax
import jax.numpy as jnp
from jax import lax
from jax.experimental import pallas as pl
from jax.experimental.pallas import tpu as pltpu
from jax.experimental.pallas import tpu_sc as plsc

N_NODES = 10000
N_PAD = 10240
D_IN = 128
D_OUT = 128
D_HALF = D_OUT // 2
N_SUBCORES = 16
N_CORES = 2
CHUNK = 128
IB = 8
ROWS_PER_S = N_PAD // N_SUBCORES
STAGE_STEPS = ROWS_PER_S // CHUNK


def _mm_body(x_ref, w_ref, o_ref):
    h = jnp.dot(x_ref[...], w_ref[...], preferred_element_type=jnp.float32)
    o_ref[0] = h[:, :D_HALF]
    o_ref[1] = h[:, D_HALF:]


def _matmul_split(x, W):
    m_blk = 1000
    grid = (x.shape[0] // m_blk,)
    return pl.pallas_call(
        _mm_body,
        grid=grid,
        in_specs=[
            pl.BlockSpec((m_blk, D_IN), lambda i: (i, 0)),
            pl.BlockSpec((D_IN, D_OUT), lambda i: (0, 0)),
        ],
        out_specs=pl.BlockSpec((N_CORES, m_blk, D_HALF), lambda i: (0, i, 0)),
        out_shape=jax.ShapeDtypeStruct((N_CORES, N_PAD, D_HALF), jnp.float32),
    )(x, W)


def _make_sc_kernel(n_blocks):
    mesh = plsc.VectorSubcoreMesh(core_axis_name="c", subcore_axis_name="s")

    @functools.partial(
        pl.kernel,
        out_type=jax.ShapeDtypeStruct((N_PAD, D_OUT), jnp.float32),
        mesh=mesh,
        scratch_types=[
            pltpu.VMEM_SHARED((N_PAD, D_HALF), jnp.float32),
            pltpu.VMEM_SHARED((N_PAD, D_HALF), jnp.float32),
            pltpu.VMEM((CHUNK, D_HALF), jnp.float32),
            pltpu.VMEM((CHUNK, D_HALF), jnp.float32),
            pltpu.VMEM((CHUNK, D_HALF), jnp.float32),
            pltpu.VMEM((CHUNK, D_HALF), jnp.float32),
            pltpu.VMEM((2, IB, CHUNK), jnp.int32),
            pltpu.VMEM((2, IB, CHUNK), jnp.int32),
            pltpu.VMEM((2, IB * CHUNK), jnp.float32),
            [pltpu.SemaphoreType.DMA] * 4,
            [pltpu.SemaphoreType.DMA] * 4,
            pltpu.SemaphoreType.DMA,
            [pltpu.SemaphoreType.DMA] * 2,
        ],
        compiler_params=pltpu.CompilerParams(
            needs_layout_passes=False, use_tc_tiling_on_sc=False),
    )
    def sc_kernel(h_hbm, src_hbm, dst_hbm, val_hbm, out_hbm,
                  h_sp, acc, buf, bufb, bufc, bufd, sidx, didx, vval,
                  gsems, ssems, isem, hsems):
        c = lax.axis_index("c")
        s = lax.axis_index("s")
        row0 = s * ROWS_PER_S
        bufs = (buf, bufb, bufc, bufd)
        h_c = h_hbm.at[c]

        @pl.loop(0, CHUNK)
        def _(r):
            for q in range(D_HALF // 16):
                buf[r, pl.ds(q * 16, 16)] = jnp.zeros((16,), jnp.float32)

        zcp = [
            pltpu.async_copy(
                buf, acc.at[pl.ds(row0 + i * CHUNK, CHUNK)], ssems[i % 4])
            for i in range(STAGE_STEPS)
        ]
        i0 = [
            pltpu.async_copy(src_hbm.at[s].at[pl.ds(0, IB)], sidx.at[0], isem),
            pltpu.async_copy(dst_hbm.at[s].at[pl.ds(0, IB)], didx.at[0], isem),
            pltpu.async_copy(
                val_hbm.at[s].at[pl.ds(0, IB * CHUNK)], vval.at[0], isem),
        ]

        def _hsl(i):
            return pl.ds(row0 + i * CHUNK, CHUNK)

        hb = (bufc, bufd)
        hi = [None] * STAGE_STEPS
        ho = [None] * STAGE_STEPS
        hi[0] = pltpu.async_copy(h_c.at[_hsl(0)], bufc, hsems[0])
        for i in range(STAGE_STEPS):
            if i + 1 < STAGE_STEPS:
                if i >= 1:
                    ho[i - 1].wait()
                hi[i + 1] = pltpu.async_copy(
                    h_c.at[_hsl(i + 1)], hb[(i + 1) % 2], hsems[(i + 1) % 2])
            hi[i].wait()
            ho[i] = pltpu.async_copy(hb[i % 2], h_sp.at[_hsl(i)], hsems[i % 2])
        ho[STAGE_STEPS - 2].wait()
        ho[STAGE_STEPS - 1].wait()

        for cp in zcp + i0:
            cp.wait()
        plsc.subcore_barrier()

        def _scale(rb, vv_ref, u):
            @pl.loop(0, CHUNK // 16, unroll=2)
            def _(g):
                vv = vv_ref[pl.ds(u * CHUNK + g * 16, 16)]
                for k in range(16):
                    vsp = jnp.broadcast_to(vv[k], (16,))
                    j = g * 16 + k
                    for q in range(D_HALF // 16):
                        sl = pl.ds(q * 16, 16)
                        rb[j, sl] = rb[j, sl] * vsp

        @pl.loop(0, n_blocks)
        def _(b):
            par = b % 2
            nxt = (b + 1) % n_blocks
            sidx_b = sidx.at[par]
            didx_b = didx.at[par]
            vval_b = vval.at[par]

            pf = [
                pltpu.async_copy(
                    src_hbm.at[s].at[pl.ds(nxt * IB, IB)],
                    sidx.at[1 - par], isem),
                pltpu.async_copy(
                    dst_hbm.at[s].at[pl.ds(nxt * IB, IB)],
                    didx.at[1 - par], isem),
                pltpu.async_copy(
                    val_hbm.at[s].at[pl.ds(nxt * IB * CHUNK, IB * CHUNK)],
                    vval.at[1 - par], isem),
            ]

            def _gather(u, i):
                return pltpu.async_copy(
                    h_sp.at[sidx_b.at[u]], bufs[i], gsems[i])

            def _scatter(u, i):
                return pltpu.async_copy(
                    bufs[i], acc.at[didx_b.at[u]], ssems[i], add=True)

            gd_ = [None] * IB
            sd_ = [None] * IB
            gd_[0] = _gather(0, 0)
            gd_[1] = _gather(1, 1)
            for u in range(IB):
                i = u % 4
                gd_[u].wait()
                _scale(bufs[i], vval_b, u)
                sd_[u] = _scatter(u, i)
                w = u + 2
                if w < IB:
                    if w >= 4:
                        sd_[w - 4].wait()
                    gd_[w] = _gather(w, w % 4)
            for u in range(IB - 4, IB):
                sd_[u].wait()
            for cp in pf:
                cp.wait()

        plsc.subcore_barrier()

        def _cin(i):
            return pltpu.async_copy(
                acc.at[pl.ds(row0 + i * CHUNK, CHUNK)], bufs[i % 2],
                gsems[i % 2])

        def _cout(i):
            return pltpu.async_copy(
                bufs[i % 2],
                out_hbm.at[pl.ds(row0 + i * CHUNK, CHUNK),
                           pl.ds(c * D_HALF, D_HALF)],
                ssems[i % 2])

        def _relu(i):
            rb = bufs[i % 2]

            @pl.loop(0, CHUNK)
            def _(r):
                for q in range(D_HALF // 16):
                    sl = pl.ds(q * 16, 16)
                    rb[r, sl] = jnp.maximum(rb[r, sl], 0.0)

        cin = [None] * STAGE_STEPS
        cout = [None] * STAGE_STEPS
        cin[0] = _cin(0)
        for i in range(STAGE_STEPS):
            if i + 1 < STAGE_STEPS:
                if i >= 1:
                    cout[i - 1].wait()
                cin[i + 1] = _cin(i + 1)
            cin[i].wait()
            _relu(i)
            cout[i] = _cout(i)
        cout[STAGE_STEPS - 2].wait()
        cout[STAGE_STEPS - 1].wait()

    return sc_kernel


def kernel(x, edge_index, adj_values, W):
    src = edge_index[0].astype(jnp.int32)
    dst = edge_index[1].astype(jnp.int32)
    val = adj_values.astype(jnp.float32)

    n_edges = src.shape[0]
    blk = N_SUBCORES * CHUNK * IB
    per_s = (-(-n_edges // blk)) * blk // N_SUBCORES
    n_blocks = per_s // (CHUNK * IB)
    pad = per_s * N_SUBCORES - n_edges
    srcp = jnp.pad(src, (0, pad)).reshape(N_SUBCORES, per_s // CHUNK, CHUNK)
    dstp = jnp.pad(dst, (0, pad)).reshape(N_SUBCORES, per_s // CHUNK, CHUNK)
    valp = jnp.pad(val, (0, pad)).reshape(N_SUBCORES, per_s)

    h2 = _matmul_split(x, W)
    o = _make_sc_kernel(n_blocks)(h2, srcp, dstp, valp)
    return o[:x.shape[0]]
